# baseline (speedup 1.0000x reference)
import functools

import jax
import jax.numpy as jnp
from jax import lax
from jax.experimental import pallas as pl
from jax.experimental.pallas import tpu as pltpu
from jax.experimental.pallas import tpu_sc as plsc

DIM = 64
LANES = 16
CHUNK = 128
NBUF = 2
N_WORKERS = 32

_GATHER_DNUMS = lax.GatherDimensionNumbers(
    offset_dims=(), collapsed_slice_dims=(0,), start_index_map=(0,))


def _bcast_lane(x16, r):
    idx = jnp.full((LANES, 1), r, jnp.int32)
    return lax.gather(x16, idx, _GATHER_DNUMS, (1,),
                      mode=lax.GatherScatterMode.PROMISE_IN_BOUNDS)


def _sc_gather(n):
    bpw = n // N_WORKERS
    nchunks = bpw // CHUNK
    nrounds = nchunks // NBUF
    obig = CHUNK // 2
    mesh = plsc.VectorSubcoreMesh(core_axis_name="c", subcore_axis_name="s")

    @functools.partial(
        pl.kernel,
        mesh=mesh,
        compiler_params=pltpu.CompilerParams(use_tc_tiling_on_sc=True,
                                             needs_layout_passes=False),
        out_type=jax.ShapeDtypeStruct((n // 2, 2 * DIM), jnp.float32),
        scratch_types=[
            pltpu.VMEM((nchunks, CHUNK), jnp.int32),
            pltpu.VMEM((nchunks, CHUNK), jnp.int32),
            pltpu.VMEM((nchunks, CHUNK), jnp.int32),
            pltpu.VMEM((NBUF, CHUNK, 2 * DIM), jnp.float32),
            pltpu.VMEM((NBUF, obig, 2 * DIM), jnp.float32),
            pltpu.SemaphoreType.DMA((NBUF,)),
            pltpu.SemaphoreType.DMA((NBUF,)),
        ],
    )
    def body(text_hbm, mask_hbm, table_hbm, out_hbm,
             text_v, mask_v, idx_v, rows_v, obuf_v, gsem, wsem):
        nc = jax.lax.axis_size("c")
        wid = lax.axis_index("s") * nc + lax.axis_index("c")
        pltpu.sync_copy(text_hbm.at[wid], text_v)
        pltpu.sync_copy(mask_hbm.at[wid], mask_v)
        obase = wid * (bpw // 2)

        def compute_chunk(j, _):
            for k in range(CHUNK // LANES):
                sl = pl.ds(k * LANES, LANES)
                idx_v[j, sl] = lax.shift_right_logical(text_v[j, sl], 1)
            return 0

        lax.fori_loop(0, nchunks, compute_chunk, 0)

        def gstart(b, j):
            pltpu.make_async_copy(
                table_hbm.at[idx_v.at[j]], rows_v.at[b], gsem.at[b]).start()

        def gwait(b, j):
            pltpu.make_async_copy(
                table_hbm.at[idx_v.at[j]], rows_v.at[b], gsem.at[b]).wait()

        def wstart(b, j):
            pltpu.make_async_copy(
                obuf_v.at[b], out_hbm.at[pl.ds(obase + j * obig, obig)],
                wsem.at[b]).start()

        def wwait(b, j):
            pltpu.make_async_copy(
                obuf_v.at[b], out_hbm.at[pl.ds(obase + j * obig, obig)],
                wsem.at[b]).wait()

        iota16 = lax.iota(jnp.int32, LANES)

        def select_rows(b, j):
            def group(g, _):
                sl = pl.ds(g * LANES, LANES)
                m16 = mask_v[j, sl].astype(jnp.float32)
                h64_16 = (text_v[j, sl] & 1) * jnp.int32(DIM)
                for r in range(LANES):
                    row = g * LANES + r
                    hb = _bcast_lane(h64_16, r)
                    mg = _bcast_lane(m16, r)
                    orow = g * (LANES // 2) + r // 2
                    p = r % 2
                    for k in range(DIM // LANES):
                        src = plsc.load_gather(
                            rows_v, [jnp.full((LANES,), b, jnp.int32),
                                     jnp.full((LANES,), row, jnp.int32),
                                     hb + (iota16 + k * LANES)])
                        obuf_v[b, orow, pl.ds(p * DIM + k * LANES, LANES)] = src * mg
                return 0

            lax.fori_loop(0, CHUNK // LANES, group, 0)

        for b in range(NBUF):
            gstart(b, b)

        def pipeline_round(r, _):
            j0 = r * NBUF
            for b in range(NBUF):
                j = j0 + b
                gwait(b, j)

                @pl.when(r > 0)
                def _():
                    wwait(b, j - NBUF)

                select_rows(b, j)
                wstart(b, j)
            jn0 = j0 + NBUF
            for b in range(NBUF):

                @pl.when(jn0 + b < nchunks)
                def _():
                    gstart(b, jn0 + b)

            return 0

        lax.fori_loop(0, nrounds, pipeline_round, 0)
        for b in range(NBUF):
            wwait(b, nchunks - NBUF + b)

    return body


def kernel(text, word_mask, embedding_matrix):
    B, L = text.shape
    n = B * L
    perw = n // (N_WORKERS * CHUNK)
    text3 = text.reshape(N_WORKERS, perw, CHUNK).astype(jnp.int32)
    mask3 = word_mask.reshape(N_WORKERS, perw, CHUNK).astype(jnp.int32)
    table2 = embedding_matrix.reshape(embedding_matrix.shape[0] // 2, 2 * DIM)
    out = _sc_gather(n)(text3, mask3, table2)
    return out.reshape(B, L, DIM)


# speedup vs baseline: 1.1209x; 1.1209x over previous
import functools

import jax
import jax.numpy as jnp
from jax import lax
from jax.experimental import pallas as pl
from jax.experimental.pallas import tpu as pltpu
from jax.experimental.pallas import tpu_sc as plsc

DIM = 64
LANES = 16

_GATHER_DNUMS = lax.GatherDimensionNumbers(
    offset_dims=(), collapsed_slice_dims=(0,), start_index_map=(0,))


def _bcast_lane(x16, r):
    idx = jnp.full((LANES, 1), r, jnp.int32)
    return lax.gather(x16, idx, _GATHER_DNUMS, (1,),
                      mode=lax.GatherScatterMode.PROMISE_IN_BOUNDS)
CHUNK = 128
NBUF = 5
N_WORKERS = 32


def _sc_gather(n):
    bpw = n // N_WORKERS
    nchunks = bpw // CHUNK
    nrounds = nchunks // NBUF
    mesh = plsc.VectorSubcoreMesh(core_axis_name="c", subcore_axis_name="s")

    @functools.partial(
        pl.kernel,
        mesh=mesh,
        compiler_params=pltpu.CompilerParams(use_tc_tiling_on_sc=False),
        out_type=jax.ShapeDtypeStruct((n, DIM), jnp.float32),
        scratch_types=[
            pltpu.VMEM((nchunks, CHUNK), jnp.int32),
            pltpu.VMEM((nchunks, CHUNK), jnp.int32),
            pltpu.VMEM((NBUF, CHUNK, DIM), jnp.float32),
            pltpu.SemaphoreType.DMA((NBUF,)),
            pltpu.SemaphoreType.DMA((NBUF,)),
        ],
    )
    def body(text_hbm, mask_hbm, table_hbm, out_hbm, idx_v, mask_v, rows_v, gsem, wsem):
        nc = jax.lax.axis_size("c")
        wid = lax.axis_index("s") * nc + lax.axis_index("c")
        base = wid * nchunks
        pltpu.sync_copy(text_hbm.at[pl.ds(base, nchunks)], idx_v)
        pltpu.sync_copy(mask_hbm.at[pl.ds(base, nchunks)], mask_v)
        rbase = wid * bpw

        def gstart(b, j):
            pltpu.make_async_copy(
                table_hbm.at[idx_v.at[j]], rows_v.at[b], gsem.at[b]).start()

        def gwait(b, j):
            pltpu.make_async_copy(
                table_hbm.at[idx_v.at[j]], rows_v.at[b], gsem.at[b]).wait()

        def wstart(b, j):
            pltpu.make_async_copy(
                rows_v.at[b], out_hbm.at[pl.ds(rbase + j * CHUNK, CHUNK)],
                wsem.at[b]).start()

        def wwait(b, j):
            pltpu.make_async_copy(
                rows_v.at[b], out_hbm.at[pl.ds(rbase + j * CHUNK, CHUNK)],
                wsem.at[b]).wait()

        def mask_rows(b, j):
            def group(g, _):
                m16 = mask_v[j, pl.ds(g * LANES, LANES)].astype(jnp.float32)
                for r in range(LANES):
                    mg = _bcast_lane(m16, r)
                    row = g * LANES + r
                    for k in range(DIM // LANES):
                        sl = pl.ds(k * LANES, LANES)
                        rows_v[b, row, sl] = rows_v[b, row, sl] * mg
                return 0

            lax.fori_loop(0, CHUNK // LANES, group, 0)

        for b in range(NBUF):
            gstart(b, b)

        def pipeline_round(r, _):
            j0 = r * NBUF
            for b in range(NBUF):
                gwait(b, j0 + b)
                mask_rows(b, j0 + b)
                wstart(b, j0 + b)
            jn0 = j0 + NBUF
            for b in range(NBUF):

                @pl.when(jn0 + b < nchunks)
                def _():
                    wwait(b, j0 + b)
                    gstart(b, jn0 + b)

            return 0

        lax.fori_loop(0, nrounds, pipeline_round, 0)
        for b in range(NBUF):
            wwait(b, nchunks - NBUF + b)

    return body


def kernel(text, word_mask, embedding_matrix):
    B, L = text.shape
    n = B * L
    text2 = text.reshape(n // CHUNK, CHUNK).astype(jnp.int32)
    mask2 = word_mask.reshape(n // CHUNK, CHUNK).astype(jnp.int32)
    out = _sc_gather(n)(text2, mask2, embedding_matrix)
    return out.reshape(B, L, DIM)
